# baseline (speedup 1.0000x reference)
import functools

import jax
import jax.numpy as jnp
from jax import lax
from jax.experimental import pallas as pl
from jax.experimental.pallas import tpu as pltpu
from jax.experimental.pallas import tpu_sc as plsc

B = 1048576
D = 7
ROWS = 95
ROWS_PAD = 96
NC, NS, L = 2, 16, 16
NW = NC * NS
B_PER_W = B // NW
CHUNK = 4096
N_CHUNKS = B_PER_W // CHUNK
VECS = CHUNK // L


def _sc_gather(table_flat, idx):
    mesh = plsc.VectorSubcoreMesh(
        core_axis_name="c", subcore_axis_name="s", num_cores=NC, num_subcores=NS
    )

    @functools.partial(
        pl.kernel,
        out_type=jax.ShapeDtypeStruct((D, B), jnp.float32),
        mesh=mesh,
        scratch_types=[
            pltpu.VMEM((ROWS_PAD * D,), jnp.float32),
            [pltpu.VMEM((ROWS_PAD,), jnp.float32) for _ in range(D)],
            pltpu.VMEM((B_PER_W,), jnp.int32),
            [pltpu.VMEM((D, CHUNK), jnp.float32) for _ in range(2)],
            [pltpu.SemaphoreType.DMA for _ in range(3)],
        ],
        compiler_params=pltpu.CompilerParams(
            needs_layout_passes=False, use_tc_tiling_on_sc=True
        ),
    )
    def k(tab_hbm, idx_hbm, out_hbm, tab_v, cols_v, idx_v, out_vs, sems):
        wid = lax.axis_index("s") * NC + lax.axis_index("c")
        base = wid * B_PER_W

        prologue = [
            pltpu.async_copy(tab_hbm, tab_v.at[pl.ds(0, ROWS * D)], sems[2]),
            pltpu.async_copy(idx_hbm.at[pl.ds(base, B_PER_W)], idx_v, sems[2]),
        ]
        for p in prologue:
            p.wait()

        lane = lax.iota(jnp.int32, L)
        for c in range(D):
            for kk in range(ROWS_PAD // L):
                zvec = lane + kk * L
                cols_v[c][pl.ds(kk * L, L)] = plsc.load_gather(
                    tab_v, [zvec * D + c]
                )

        descs = [None, None]
        for s in range(N_CHUNKS):
            b = s % 2
            if descs[b] is not None:
                descs[b].wait()

            @plsc.parallel_loop(0, VECS, unroll=1)
            def vec_body(i, _s=s, _b=b):
                z = idx_v[pl.ds(_s * CHUNK + i * L, L)]
                for c in range(D):
                    out_vs[_b][c, pl.ds(i * L, L)] = plsc.load_gather(
                        cols_v[c], [z]
                    )

            descs[b] = pltpu.async_copy(
                out_vs[b], out_hbm.at[:, pl.ds(base + s * CHUNK, CHUNK)], sems[b]
            )
        for d in descs:
            d.wait()

    return k(table_flat, idx)


def kernel(atomic_numbers, table):
    idx = atomic_numbers.astype(jnp.int32)
    return _sc_gather(table.reshape(ROWS * D), idx).T


# speedup vs baseline: 1.0118x; 1.0118x over previous
import functools

import jax
import jax.numpy as jnp
from jax import lax
from jax.experimental import pallas as pl
from jax.experimental.pallas import tpu as pltpu
from jax.experimental.pallas import tpu_sc as plsc

B = 1048576
D = 7
ROWS = 95
ROWS_PAD = 96
NC, NS, L = 2, 16, 16
NW = NC * NS
B_PER_W = B // NW
CHUNK = 4096
N_CHUNKS = B_PER_W // CHUNK
VECS = CHUNK // L


def _sc_gather(table_cols, idx):
    mesh = plsc.VectorSubcoreMesh(
        core_axis_name="c", subcore_axis_name="s", num_cores=NC, num_subcores=NS
    )

    @functools.partial(
        pl.kernel,
        out_type=jax.ShapeDtypeStruct((D, B), jnp.float32),
        mesh=mesh,
        scratch_types=[
            [pltpu.VMEM((ROWS_PAD,), jnp.float32) for _ in range(D)],
            pltpu.VMEM((B_PER_W,), jnp.int32),
            [pltpu.VMEM((D, CHUNK), jnp.float32) for _ in range(2)],
            [pltpu.SemaphoreType.DMA for _ in range(3)],
        ],
        compiler_params=pltpu.CompilerParams(
            needs_layout_passes=False, use_tc_tiling_on_sc=True
        ),
    )
    def k(tab_hbm, idx_hbm, out_hbm, cols_v, idx_v, out_vs, sems):
        wid = lax.axis_index("s") * NC + lax.axis_index("c")
        base = wid * B_PER_W

        prologue = [
            pltpu.async_copy(
                tab_hbm.at[pl.ds(c * ROWS_PAD, ROWS_PAD)], cols_v[c], sems[2]
            )
            for c in range(D)
        ]
        prologue.append(
            pltpu.async_copy(idx_hbm.at[pl.ds(base, B_PER_W)], idx_v, sems[2])
        )
        for p in prologue:
            p.wait()

        descs = [None, None]
        for s in range(N_CHUNKS):
            b = s % 2
            if descs[b] is not None:
                descs[b].wait()

            @plsc.parallel_loop(0, VECS, unroll=1)
            def vec_body(i, _s=s, _b=b):
                z = idx_v[pl.ds(_s * CHUNK + i * L, L)]
                for c in range(D):
                    out_vs[_b][c, pl.ds(i * L, L)] = plsc.load_gather(
                        cols_v[c], [z]
                    )

            descs[b] = pltpu.async_copy(
                out_vs[b], out_hbm.at[:, pl.ds(base + s * CHUNK, CHUNK)], sems[b]
            )
        for d in descs:
            d.wait()

    return k(table_cols, idx)


def kernel(atomic_numbers, table):
    idx = atomic_numbers.astype(jnp.int32)
    cols = (
        jnp.zeros((D, ROWS_PAD), jnp.float32).at[:, :ROWS].set(table.T).reshape(-1)
    )
    return _sc_gather(cols, idx).T
